# baseline (speedup 1.0000x reference)
import jax
import jax.numpy as jnp
from jax.experimental import pallas as pl

_N = 262144
_ROWS = 512
_COLS = 512


def _ghm_kernel(pred_ref, target_ref, dens_ref, out_ref):
    p = pred_ref[...]
    t = target_ref[...]
    diff = p - t
    g = jnp.abs(diff)
    gmax = jnp.max(g)
    bins = dens_ref.shape[-1]
    scaled = g / gmax * (bins - 1)
    mse = diff * diff
    zero = jnp.zeros_like(mse)
    total = jnp.sum(mse)
    w_prev = 1.0 / (dens_ref[0, 0] + 1e-6)
    loss = w_prev * total
    for b in range(1, bins):
        w_b = 1.0 / (dens_ref[0, b] + 1e-6)
        u_b = jnp.sum(jnp.where(scaled >= b, mse, zero))
        loss = loss + (w_b - w_prev) * u_b
        w_prev = w_b
    out_ref[...] = jnp.full((1, 1), loss * (1.0 / _N), dtype=jnp.float32)


def kernel(pred, target, gradient_hist, grad_density):
    del gradient_hist
    p2 = pred.reshape(_ROWS, _COLS)
    t2 = target.reshape(_ROWS, _COLS)
    d2 = grad_density.reshape(1, -1)
    out = pl.pallas_call(
        _ghm_kernel,
        out_shape=jax.ShapeDtypeStruct((1, 1), jnp.float32),
    )(p2, t2, d2)
    return out[0, 0]
